# Initial kernel scaffold; baseline (speedup 1.0000x reference)
import jax
import jax.numpy as jnp
from jax.experimental import pallas as pl


def kernel(net, inp, corr, flow, ii, jj, kk, params):
    raise NotImplementedError("write your pallas kernel here")


# speedup vs baseline: 2.2300x; 2.2300x over previous
import functools

import jax
import jax.numpy as jnp
from jax import lax
from jax.experimental import pallas as pl
from jax.experimental.pallas import tpu as pltpu
from jax.experimental.pallas import tpu_sc as plsc

E = 40000
D = 384
CD = 882
NK = 4000
NF = 120
F = NF + 2
NW = 32
SLOTS_W = 15256
TPAD = NW * SLOTS_W
LOCPAD = 15264

BE = 1000
GB = 160
NCHE = E // GB
ITER_W = (NCHE + NW - 1) // NW

SEGC = 2048
PADR = 48
ZR = SEGC // 16
GBS = 64
NCHS = E // GBS
ITER_S = (NCHS + NW - 1) // NW


def _f32(x):
    return x.astype(jnp.float32)


def _ln(x, g, b, eps=1e-3):
    mu = jnp.mean(x, axis=-1, keepdims=True)
    var = jnp.mean((x - mu) ** 2, axis=-1, keepdims=True)
    return (x - mu) / jnp.sqrt(var + eps) * g + b


def _mm(x, w):
    return jnp.dot(x, w, preferred_element_type=jnp.float32)


def _relu(x):
    return jnp.maximum(x, 0.0)


def _full(shape):
    return pl.BlockSpec(shape, lambda i: tuple(0 for _ in shape))


def _eblk(d):
    return pl.BlockSpec((BE, d), lambda i: (i, 0))


def _tc_embed(net, inp, corr, p):
    c = p["corr"]

    def body(net_r, inp_r, corr_r, w1, b1, w2, b2, lg, lb, w3, b3, ng, nb, o_r):
        h = _relu(_mm(corr_r[...], w1[...]) + b1[...])
        h = _mm(h, w2[...]) + b2[...]
        h = _relu(_ln(h, lg[...], lb[...]))
        ce = _mm(h, w3[...]) + b3[...]
        o_r[...] = _ln(net_r[...] + inp_r[...] + ce, ng[...], nb[...])

    return pl.pallas_call(
        body,
        grid=(E // BE,),
        in_specs=[_eblk(D), _eblk(D), _eblk(CD),
                  _full((CD, D)), _full((1, D)), _full((D, D)), _full((1, D)),
                  _full((1, D)), _full((1, D)), _full((D, D)), _full((1, D)),
                  _full((1, D)), _full((1, D))],
        out_specs=_eblk(D),
        out_shape=jax.ShapeDtypeStruct((E, D), jnp.float32),
    )(net, inp, corr,
      c["l1"]["w"].T, c["l1"]["b"][None], c["l2"]["w"].T, c["l2"]["b"][None],
      c["ln"]["g"][None], c["ln"]["b"][None], c["l3"]["w"].T, c["l3"]["b"][None],
      p["norm"]["g"][None], p["norm"]["b"][None])


def _tc_resid(x, g1, g2, p):
    c1a, c1b = p["c1"]
    c2a, c2b = p["c2"]

    def body(x_r, g1_r, g2_r, aw1, ab1, aw2, ab2, bw1, bb1, bw2, bb2, o_r):
        r1 = _mm(_relu(_mm(g1_r[...], aw1[...]) + ab1[...]), aw2[...]) + ab2[...]
        r2 = _mm(_relu(_mm(g2_r[...], bw1[...]) + bb1[...]), bw2[...]) + bb2[...]
        o_r[...] = x_r[...] + r1 + r2

    return pl.pallas_call(
        body,
        grid=(E // BE,),
        in_specs=[_eblk(D), _eblk(D), _eblk(D)]
        + [_full((D, D)), _full((1, D))] * 4,
        out_specs=_eblk(D),
        out_shape=jax.ShapeDtypeStruct((E, D), jnp.float32),
    )(x, g1, g2,
      c1a["w"].T, c1a["b"][None], c1b["w"].T, c1b["b"][None],
      c2a["w"].T, c2a["b"][None], c2b["w"].T, c2b["b"][None])


def _tc_softprep(x, hg, agg):
    add_h = hg is not None

    def body(*refs):
        if add_h:
            x_r, hg_r, fw, fb, gw, gb, x3_r, ex_r, p_r = refs
            x3 = x_r[...] + hg_r[...]
            x3_r[...] = x3
        else:
            x_r, fw, fb, gw, gb, ex_r, p_r = refs
            x3 = x_r[...]
        ex = jnp.exp(_mm(x3, gw[...]) + gb[...])
        ex_r[...] = ex
        p_r[...] = (_mm(x3, fw[...]) + fb[...]) * ex

    n_out = 3 if add_h else 2
    ins = [x, hg] if add_h else [x]
    return pl.pallas_call(
        body,
        grid=(E // BE,),
        in_specs=[_eblk(D)] * len(ins)
        + [_full((D, D)), _full((1, D)), _full((D, D)), _full((1, D))],
        out_specs=[_eblk(D)] * n_out,
        out_shape=[jax.ShapeDtypeStruct((E, D), jnp.float32)] * n_out,
    )(*ins, agg["f"]["w"].T, agg["f"]["b"][None],
      agg["g"]["w"].T, agg["g"]["b"][None])


def _tc_seg_h(parts, agg):
    R = parts.shape[1]
    BD = 512

    def body(p_r, hw, hb, o_r):
        blk = p_r[...]
        es = blk[0]
        ps = blk[1]
        y = ps / jnp.where(es == 0.0, 1.0, es)
        o_r[...] = _mm(y, hw[...]) + hb[...]

    return pl.pallas_call(
        body,
        grid=(R // BD,),
        in_specs=[pl.BlockSpec((2, BD, D), lambda i: (0, i, 0)),
                  _full((D, D)), _full((1, D))],
        out_specs=pl.BlockSpec((BD, D), lambda i: (i, 0)),
        out_shape=jax.ShapeDtypeStruct((R, D), jnp.float32),
    )(parts, agg["h"]["w"].T, agg["h"]["b"][None])


def _tc_final(x, hg, p):
    g = p["gru"]
    g1, g2 = g["gr1"], g["gr2"]

    def body(x_r, hg_r, l1g, l1b, g1g, g1gb, g1r1, g1r1b, g1r2, g1r2b,
             l2g, l2b, g2g, g2gb, g2r1, g2r1b, g2r2, g2r2b,
             dw, db, ww, wb, net_r, d_r, w_r):
        n = _ln(x_r[...] + hg_r[...], l1g[...], l1b[...])
        gate = jax.nn.sigmoid(_mm(n, g1g[...]) + g1gb[...])
        res = _mm(_relu(_mm(n, g1r1[...]) + g1r1b[...]), g1r2[...]) + g1r2b[...]
        n = n * gate + res
        n = _ln(n, l2g[...], l2b[...])
        gate = jax.nn.sigmoid(_mm(n, g2g[...]) + g2gb[...])
        res = _mm(_relu(_mm(n, g2r1[...]) + g2r1b[...]), g2r2[...]) + g2r2b[...]
        n = n * gate + res
        net_r[...] = n
        r = _relu(n)
        d_r[...] = _mm(r, dw[...]) + db[...]
        w_r[...] = jax.nn.sigmoid(_mm(r, ww[...]) + wb[...])

    return pl.pallas_call(
        body,
        grid=(E // BE,),
        in_specs=[_eblk(D), _eblk(D),
                  _full((1, D)), _full((1, D)),
                  _full((D, D)), _full((1, D)), _full((D, D)), _full((1, D)),
                  _full((D, D)), _full((1, D)),
                  _full((1, D)), _full((1, D)),
                  _full((D, D)), _full((1, D)), _full((D, D)), _full((1, D)),
                  _full((D, D)), _full((1, D)),
                  _full((D, 2)), _full((1, 2)), _full((D, 2)), _full((1, 2))],
        out_specs=[_eblk(D), pl.BlockSpec((BE, 2), lambda i: (i, 0)),
                   pl.BlockSpec((BE, 2), lambda i: (i, 0))],
        out_shape=[jax.ShapeDtypeStruct((E, D), jnp.float32),
                   jax.ShapeDtypeStruct((E, 2), jnp.float32),
                   jax.ShapeDtypeStruct((E, 2), jnp.float32)],
    )(x, hg,
      g["ln1"]["g"][None], g["ln1"]["b"][None],
      g1["gate"]["w"].T, g1["gate"]["b"][None],
      g1["res1"]["w"].T, g1["res1"]["b"][None],
      g1["res2"]["w"].T, g1["res2"]["b"][None],
      g["ln2"]["g"][None], g["ln2"]["b"][None],
      g2["gate"]["w"].T, g2["gate"]["b"][None],
      g2["res1"]["w"].T, g2["res1"]["b"][None],
      g2["res2"]["w"].T, g2["res2"]["b"][None],
      p["d"]["w"].T, p["d"]["b"][None],
      p["w"]["w"].T, p["w"]["b"][None])


def _mesh():
    return plsc.VectorSubcoreMesh(core_axis_name="c", subcore_axis_name="s")


def _wid():
    return lax.axis_index("s") * 2 + lax.axis_index("c")


def _iota16():
    return lax.broadcasted_iota(jnp.int32, (16,), 0)


def _sc_neigh_table(kk, jj):
    CH = 2000

    @functools.partial(
        pl.kernel, mesh=_mesh(),
        out_type=jax.ShapeDtypeStruct((TPAD,), jnp.int32),
        scratch_types=[pltpu.VMEM((CH,), jnp.int32),
                       pltpu.VMEM((CH,), jnp.int32),
                       pltpu.VMEM((LOCPAD,), jnp.int32)],
        compiler_params=pltpu.CompilerParams(needs_layout_passes=False),
    )
    def k(kk_h, jj_h, tab_h, kkb, jjb, loc):
        base = _wid() * SLOTS_W

        def init(i, _):
            loc[pl.ds(i * 16, 16)] = jnp.full((16,), -1, jnp.int32)
            return 0
        lax.fori_loop(0, LOCPAD // 16, init, 0)

        def chunk(ci, _):
            pltpu.sync_copy(kk_h.at[pl.ds(ci * CH, CH)], kkb)
            pltpu.sync_copy(jj_h.at[pl.ds(ci * CH, CH)], jjb)

            def grp(gi, _):
                kv = kkb[pl.ds(gi * 16, 16)]
                jv = jjb[pl.ds(gi * 16, 16)]
                lo = kv * F + jv + 1 - base
                valid = (lo >= 0) & (lo < SLOTS_W)
                loc_c = jnp.clip(lo, 0, SLOTS_W - 1)
                ev = ci * CH + gi * 16 + _iota16()
                plsc.store_scatter(loc, [loc_c], ev, mask=valid)
                return 0
            lax.fori_loop(0, CH // 16, grp, 0)
            return 0
        lax.fori_loop(0, E // CH, chunk, 0)
        pltpu.sync_copy(loc.at[pl.ds(0, SLOTS_W)], tab_h.at[pl.ds(base, SLOTS_W)])

    return k(kk, jj)


def _sc_neigh_query(kk, jj, tab):
    @functools.partial(
        pl.kernel, mesh=_mesh(),
        out_type=[jax.ShapeDtypeStruct((E,), jnp.int32),
                  jax.ShapeDtypeStruct((E,), jnp.int32)],
        scratch_types=[pltpu.VMEM((GB,), jnp.int32),
                       pltpu.VMEM((GB,), jnp.int32),
                       pltpu.VMEM((GB,), jnp.int32),
                       pltpu.VMEM((GB,), jnp.int32),
                       pltpu.SemaphoreType.DMA],
    )
    def k(kk_h, jj_h, tab_h, ixg_h, jxg_h, kkb, jjb, qb, rb, sem):
        w = _wid()

        def it(t, _):
            ci = w + t * NW

            @pl.when(ci < NCHE)
            def _():
                eb = ci * GB
                pltpu.sync_copy(kk_h.at[pl.ds(eb, GB)], kkb)
                pltpu.sync_copy(jj_h.at[pl.ds(eb, GB)], jjb)
                for off, out_h in ((0, ixg_h), (2, jxg_h)):
                    def mkq(gi, _):
                        kv = kkb[pl.ds(gi * 16, 16)]
                        jv = jjb[pl.ds(gi * 16, 16)]
                        qb[pl.ds(gi * 16, 16)] = kv * F + jv + off
                        return 0
                    lax.fori_loop(0, GB // 16, mkq, 0)
                    pltpu.async_copy(tab_h.at[qb], rb, sem).wait()

                    def remap(gi, _):
                        v = rb[pl.ds(gi * 16, 16)]
                        rb[pl.ds(gi * 16, 16)] = jnp.where(
                            v >= 0, v, E + _iota16())
                        return 0
                    lax.fori_loop(0, GB // 16, remap, 0)
                    pltpu.sync_copy(rb, out_h.at[pl.ds(eb, GB)])
            return 0
        lax.fori_loop(0, ITER_W, it, 0)

    return k(kk, jj, tab)


def _sc_gather_rows(src, idx):
    @functools.partial(
        pl.kernel, mesh=_mesh(),
        out_type=jax.ShapeDtypeStruct((E, D), jnp.float32),
        scratch_types=[pltpu.VMEM((GB,), jnp.int32),
                       pltpu.VMEM((GB, D), jnp.float32),
                       pltpu.SemaphoreType.DMA],
    )
    def k(src_h, idx_h, out_h, ib, rows, sem):
        w = _wid()

        def it(t, _):
            ci = w + t * NW

            @pl.when(ci < NCHE)
            def _():
                eb = ci * GB
                pltpu.sync_copy(idx_h.at[pl.ds(eb, GB)], ib)
                pltpu.async_copy(src_h.at[ib], rows, sem).wait()
                pltpu.sync_copy(rows, out_h.at[pl.ds(eb, GB)])
            return 0
        lax.fori_loop(0, ITER_W, it, 0)

    return k(src, idx)


def _sc_segment_sums(seg, ex, pv, ch_size, nchunks, out_rows):
    H = ch_size // NW
    HP = ((H + 15) // 16) * 16
    TROWS = H + 16
    SW = 2000
    NWIN = E // SW
    CAP = SW + 112
    FB = 64
    padbase = nchunks * ch_size

    @functools.partial(
        pl.kernel, mesh=_mesh(),
        out_type=jax.ShapeDtypeStruct((2 * out_rows, D), jnp.float32),
        scratch_types=[pltpu.VMEM((SW,), jnp.int32),
                       pltpu.VMEM((CAP,), jnp.int32),
                       pltpu.VMEM((CAP,), jnp.int32),
                       pltpu.VMEM((FB,), jnp.int32),
                       pltpu.VMEM((FB + 16,), jnp.int32),
                       pltpu.VMEM((FB, D), jnp.float32),
                       pltpu.VMEM((TROWS, D), jnp.float32),
                       pltpu.VMEM((HP,), jnp.int32),
                       pltpu.SemaphoreType.DMA],
        compiler_params=pltpu.CompilerParams(needs_layout_passes=False),
    )
    def k(seg_h, ex_h, pv_h, out_h, segw, eidx, lrow, gbuf, lbuf, rbuf,
          tab, idxout, sem):
        w = _wid()
        iota = _iota16()

        for a, val_h in ((0, ex_h), (1, pv_h)):
            def per_chunk(ch, _):
                base = ch * ch_size

                def zrow(r, _):
                    for c in range(D // 16):
                        tab[r, pl.ds(c * 16, 16)] = jnp.zeros((16,), jnp.float32)
                    return 0
                lax.fori_loop(0, TROWS, zrow, 0)

                def win(wi, _):
                    pltpu.sync_copy(seg_h.at[pl.ds(wi * SW, SW)], segw)

                    def grp(gi, off):
                        sv = segw[pl.ds(gi * 16, 16)]
                        lo = sv - base
                        mine = (lo >= 0) & (lo < ch_size) & (sv % NW == w)
                        lr = jnp.clip(lo, 0, ch_size - 1) // NW
                        ev = wi * SW + gi * 16 + iota
                        plsc.store_compressed(eidx.at[pl.ds(off, 16)], ev, mask=mine)
                        plsc.store_compressed(lrow.at[pl.ds(off, 16)], lr, mask=mine)
                        return off + jnp.sum(mine.astype(jnp.int32))
                    off = lax.fori_loop(0, SW // 16, grp, 0)

                    ones = jnp.ones((16,), jnp.bool_)
                    for pg in range(4):
                        plsc.store_compressed(eidx.at[pl.ds(off + pg * 16, 16)],
                                              iota, mask=ones)
                        plsc.store_compressed(lrow.at[pl.ds(off + pg * 16, 16)],
                                              H + iota, mask=ones)

                    def flush(b, _):
                        @pl.when(b * FB < off)
                        def _():
                            def cp(i, _):
                                gbuf[pl.ds(i * 16, 16)] = eidx[
                                    pl.ds(b * FB + i * 16, 16)]
                                lbuf[pl.ds(i * 16, 16)] = lrow[
                                    pl.ds(b * FB + i * 16, 16)]
                                return 0
                            lax.fori_loop(0, FB // 16, cp, 0)
                            pltpu.async_copy(val_h.at[gbuf], rbuf, sem).wait()

                            def acc(r, _):
                                s = lbuf[pl.ds(r, 16)][0]
                                for dpart in range(D // 16):
                                    plsc.addupdate(
                                        tab.at[s, pl.ds(dpart * 16, 16)],
                                        rbuf[r, pl.ds(dpart * 16, 16)])
                                return 0
                            lax.fori_loop(0, FB, acc, 0)
                        return 0
                    lax.fori_loop(0, SW // FB, flush, 0)
                    return 0
                lax.fori_loop(0, NWIN, win, 0)

                def mkout(gi, _):
                    j = gi * 16 + iota
                    inr = j < H
                    oidx = jnp.where(
                        inr, a * out_rows + base + j * NW + w,
                        a * out_rows + padbase + (w % 6) * 16
                        + jnp.clip(j - H, 0, 15))
                    idxout[pl.ds(gi * 16, 16)] = oidx
                    return 0
                lax.fori_loop(0, HP // 16, mkout, 0)
                pltpu.sync_copy(tab.at[pl.ds(0, HP)], out_h.at[idxout])
                return 0
            lax.fori_loop(0, nchunks, per_chunk, 0)

    out = k(seg, ex, pv)
    return out.reshape(2, out_rows, D)


def kernel(net, inp, corr, flow, ii, jj, kk, params):
    del flow
    net2, inp2, corr2 = net[0], inp[0], corr[0]

    x = _tc_embed(net2, inp2, corr2, params)

    tab = _sc_neigh_table(kk, jj)
    ixg, jxg = _sc_neigh_query(kk, jj, tab)
    x_pad = jnp.concatenate([x, jnp.zeros((16, D), jnp.float32)], axis=0)
    gx1 = _sc_gather_rows(x_pad, ixg)
    gx2 = _sc_gather_rows(x_pad, jxg)
    x2 = _tc_resid(x, gx1, gx2, params)

    ex, p = _tc_softprep(x2, None, params["agg_kk"])
    parts = _sc_segment_sums(kk, ex, p, 4000, 1, 4096)
    h_kk = _tc_seg_h(parts, params["agg_kk"])
    hg1 = _sc_gather_rows(h_kk, kk)

    seg_ij = ii * NF + jj
    x3, ex2, p2 = _tc_softprep(x2, hg1, params["agg_ij"])
    parts2 = _sc_segment_sums(seg_ij, ex2, p2, 7200, 2, 14848)
    h_ij = _tc_seg_h(parts2, params["agg_ij"])
    hg2 = _sc_gather_rows(h_ij, seg_ij)

    net_o, d_o, w_o = _tc_final(x3, hg2, params)
    return net_o[None], d_o[None], w_o[None]


# speedup vs baseline: 2.4339x; 1.0914x over previous
import functools

import jax
import jax.numpy as jnp
from jax import lax
from jax.experimental import pallas as pl
from jax.experimental.pallas import tpu as pltpu
from jax.experimental.pallas import tpu_sc as plsc

E = 40000
D = 384
CD = 882
NK = 4000
NF = 120
F = NF + 2
NW = 32
SLOTS_W = 15256
TPAD = NW * SLOTS_W
LOCPAD = 15264

BE = 1000
GB = 160
NCHE = E // GB
ITER_W = (NCHE + NW - 1) // NW

SEGC = 2048
PADR = 48
ZR = SEGC // 16
GBS = 64
NCHS = E // GBS
ITER_S = (NCHS + NW - 1) // NW


def _f32(x):
    return x.astype(jnp.float32)


def _ln(x, g, b, eps=1e-3):
    mu = jnp.mean(x, axis=-1, keepdims=True)
    var = jnp.mean((x - mu) ** 2, axis=-1, keepdims=True)
    return (x - mu) / jnp.sqrt(var + eps) * g + b


def _mm(x, w):
    return jnp.dot(x, w, preferred_element_type=jnp.float32)


def _relu(x):
    return jnp.maximum(x, 0.0)


def _full(shape):
    return pl.BlockSpec(shape, lambda i: tuple(0 for _ in shape))


def _eblk(d):
    return pl.BlockSpec((BE, d), lambda i: (i, 0))


def _tc_embed(net, inp, corr, p):
    c = p["corr"]

    def body(net_r, inp_r, corr_r, w1, b1, w2, b2, lg, lb, w3, b3, ng, nb, o_r):
        h = _relu(_mm(corr_r[...], w1[...]) + b1[...])
        h = _mm(h, w2[...]) + b2[...]
        h = _relu(_ln(h, lg[...], lb[...]))
        ce = _mm(h, w3[...]) + b3[...]
        o_r[...] = _ln(net_r[...] + inp_r[...] + ce, ng[...], nb[...])

    return pl.pallas_call(
        body,
        grid=(E // BE,),
        in_specs=[_eblk(D), _eblk(D), _eblk(CD),
                  _full((CD, D)), _full((1, D)), _full((D, D)), _full((1, D)),
                  _full((1, D)), _full((1, D)), _full((D, D)), _full((1, D)),
                  _full((1, D)), _full((1, D))],
        out_specs=_eblk(D),
        out_shape=jax.ShapeDtypeStruct((E, D), jnp.float32),
    )(net, inp, corr,
      c["l1"]["w"].T, c["l1"]["b"][None], c["l2"]["w"].T, c["l2"]["b"][None],
      c["ln"]["g"][None], c["ln"]["b"][None], c["l3"]["w"].T, c["l3"]["b"][None],
      p["norm"]["g"][None], p["norm"]["b"][None])


def _tc_resid(x, g1, g2, m1, m2, p):
    c1a, c1b = p["c1"]
    c2a, c2b = p["c2"]

    def body(x_r, g1_r, g2_r, m1_r, m2_r,
             aw1, ab1, aw2, ab2, bw1, bb1, bw2, bb2, o_r):
        a1 = g1_r[...] * m1_r[...]
        a2 = g2_r[...] * m2_r[...]
        r1 = _mm(_relu(_mm(a1, aw1[...]) + ab1[...]), aw2[...]) + ab2[...]
        r2 = _mm(_relu(_mm(a2, bw1[...]) + bb1[...]), bw2[...]) + bb2[...]
        o_r[...] = x_r[...] + r1 + r2

    return pl.pallas_call(
        body,
        grid=(E // BE,),
        in_specs=[_eblk(D), _eblk(D), _eblk(D), _eblk(1), _eblk(1)]
        + [_full((D, D)), _full((1, D))] * 4,
        out_specs=_eblk(D),
        out_shape=jax.ShapeDtypeStruct((E, D), jnp.float32),
    )(x, g1, g2, m1[:, None], m2[:, None],
      c1a["w"].T, c1a["b"][None], c1b["w"].T, c1b["b"][None],
      c2a["w"].T, c2a["b"][None], c2b["w"].T, c2b["b"][None])


def _tc_softprep(x, hg, agg):
    add_h = hg is not None

    def body(*refs):
        if add_h:
            x_r, hg_r, fw, fb, gw, gb, x3_r, ex_r, p_r = refs
            x3 = x_r[...] + hg_r[...]
            x3_r[...] = x3
        else:
            x_r, fw, fb, gw, gb, ex_r, p_r = refs
            x3 = x_r[...]
        ex = jnp.exp(_mm(x3, gw[...]) + gb[...])
        ex_r[...] = ex
        p_r[...] = (_mm(x3, fw[...]) + fb[...]) * ex

    n_out = 3 if add_h else 2
    ins = [x, hg] if add_h else [x]
    return pl.pallas_call(
        body,
        grid=(E // BE,),
        in_specs=[_eblk(D)] * len(ins)
        + [_full((D, D)), _full((1, D)), _full((D, D)), _full((1, D))],
        out_specs=[_eblk(D)] * n_out,
        out_shape=[jax.ShapeDtypeStruct((E, D), jnp.float32)] * n_out,
    )(*ins, agg["f"]["w"].T, agg["f"]["b"][None],
      agg["g"]["w"].T, agg["g"]["b"][None])


def _tc_seg_h(parts, agg):
    R = parts.shape[1]
    BD = 512

    def body(p_r, hw, hb, o_r):
        blk = p_r[...]
        es = blk[0]
        ps = blk[1]
        y = ps / jnp.where(es == 0.0, 1.0, es)
        o_r[...] = _mm(y, hw[...]) + hb[...]

    return pl.pallas_call(
        body,
        grid=(R // BD,),
        in_specs=[pl.BlockSpec((2, BD, D), lambda i: (0, i, 0)),
                  _full((D, D)), _full((1, D))],
        out_specs=pl.BlockSpec((BD, D), lambda i: (i, 0)),
        out_shape=jax.ShapeDtypeStruct((R, D), jnp.float32),
    )(parts, agg["h"]["w"].T, agg["h"]["b"][None])


def _tc_final(x, hg, p):
    g = p["gru"]
    g1, g2 = g["gr1"], g["gr2"]

    def body(x_r, hg_r, l1g, l1b, g1g, g1gb, g1r1, g1r1b, g1r2, g1r2b,
             l2g, l2b, g2g, g2gb, g2r1, g2r1b, g2r2, g2r2b,
             dw, db, ww, wb, net_r, d_r, w_r):
        n = _ln(x_r[...] + hg_r[...], l1g[...], l1b[...])
        gate = jax.nn.sigmoid(_mm(n, g1g[...]) + g1gb[...])
        res = _mm(_relu(_mm(n, g1r1[...]) + g1r1b[...]), g1r2[...]) + g1r2b[...]
        n = n * gate + res
        n = _ln(n, l2g[...], l2b[...])
        gate = jax.nn.sigmoid(_mm(n, g2g[...]) + g2gb[...])
        res = _mm(_relu(_mm(n, g2r1[...]) + g2r1b[...]), g2r2[...]) + g2r2b[...]
        n = n * gate + res
        net_r[...] = n
        r = _relu(n)
        d_r[...] = _mm(r, dw[...]) + db[...]
        w_r[...] = jax.nn.sigmoid(_mm(r, ww[...]) + wb[...])

    return pl.pallas_call(
        body,
        grid=(E // BE,),
        in_specs=[_eblk(D), _eblk(D),
                  _full((1, D)), _full((1, D)),
                  _full((D, D)), _full((1, D)), _full((D, D)), _full((1, D)),
                  _full((D, D)), _full((1, D)),
                  _full((1, D)), _full((1, D)),
                  _full((D, D)), _full((1, D)), _full((D, D)), _full((1, D)),
                  _full((D, D)), _full((1, D)),
                  _full((D, 2)), _full((1, 2)), _full((D, 2)), _full((1, 2))],
        out_specs=[_eblk(D), pl.BlockSpec((BE, 2), lambda i: (i, 0)),
                   pl.BlockSpec((BE, 2), lambda i: (i, 0))],
        out_shape=[jax.ShapeDtypeStruct((E, D), jnp.float32),
                   jax.ShapeDtypeStruct((E, 2), jnp.float32),
                   jax.ShapeDtypeStruct((E, 2), jnp.float32)],
    )(x, hg,
      g["ln1"]["g"][None], g["ln1"]["b"][None],
      g1["gate"]["w"].T, g1["gate"]["b"][None],
      g1["res1"]["w"].T, g1["res1"]["b"][None],
      g1["res2"]["w"].T, g1["res2"]["b"][None],
      g["ln2"]["g"][None], g["ln2"]["b"][None],
      g2["gate"]["w"].T, g2["gate"]["b"][None],
      g2["res1"]["w"].T, g2["res1"]["b"][None],
      g2["res2"]["w"].T, g2["res2"]["b"][None],
      p["d"]["w"].T, p["d"]["b"][None],
      p["w"]["w"].T, p["w"]["b"][None])


def _mesh():
    return plsc.VectorSubcoreMesh(core_axis_name="c", subcore_axis_name="s")


def _wid():
    return lax.axis_index("s") * 2 + lax.axis_index("c")


def _iota16():
    return lax.broadcasted_iota(jnp.int32, (16,), 0)


def _sc_neigh_table(kk, jj):
    CH = 2000

    @functools.partial(
        pl.kernel, mesh=_mesh(),
        out_type=jax.ShapeDtypeStruct((TPAD,), jnp.int32),
        scratch_types=[pltpu.VMEM((CH,), jnp.int32),
                       pltpu.VMEM((CH,), jnp.int32),
                       pltpu.VMEM((LOCPAD,), jnp.int32)],
        compiler_params=pltpu.CompilerParams(needs_layout_passes=False),
    )
    def k(kk_h, jj_h, tab_h, kkb, jjb, loc):
        base = _wid() * SLOTS_W

        def init(i, _):
            loc[pl.ds(i * 16, 16)] = jnp.full((16,), -1, jnp.int32)
            return 0
        lax.fori_loop(0, LOCPAD // 16, init, 0)

        def chunk(ci, _):
            pltpu.sync_copy(kk_h.at[pl.ds(ci * CH, CH)], kkb)
            pltpu.sync_copy(jj_h.at[pl.ds(ci * CH, CH)], jjb)

            def grp(gi, _):
                kv = kkb[pl.ds(gi * 16, 16)]
                jv = jjb[pl.ds(gi * 16, 16)]
                lo = kv * F + jv + 1 - base
                valid = (lo >= 0) & (lo < SLOTS_W)
                loc_c = jnp.clip(lo, 0, SLOTS_W - 1)
                ev = ci * CH + gi * 16 + _iota16()
                plsc.store_scatter(loc, [loc_c], ev, mask=valid)
                return 0
            lax.fori_loop(0, CH // 16, grp, 0)
            return 0
        lax.fori_loop(0, E // CH, chunk, 0)
        pltpu.sync_copy(loc.at[pl.ds(0, SLOTS_W)], tab_h.at[pl.ds(base, SLOTS_W)])

    return k(kk, jj)


def _sc_neigh_query(kk, jj, tab):
    @functools.partial(
        pl.kernel, mesh=_mesh(),
        out_type=[jax.ShapeDtypeStruct((E,), jnp.int32),
                  jax.ShapeDtypeStruct((E,), jnp.int32),
                  jax.ShapeDtypeStruct((E,), jnp.float32),
                  jax.ShapeDtypeStruct((E,), jnp.float32)],
        scratch_types=[pltpu.VMEM((GB,), jnp.int32),
                       pltpu.VMEM((GB,), jnp.int32),
                       pltpu.VMEM((GB,), jnp.int32),
                       pltpu.VMEM((GB,), jnp.int32),
                       pltpu.VMEM((GB,), jnp.float32),
                       pltpu.SemaphoreType.DMA],
    )
    def k(kk_h, jj_h, tab_h, ixg_h, jxg_h, m1_h, m2_h, kkb, jjb, qb, rb,
          mb, sem):
        w = _wid()

        def it(t, _):
            ci = w + t * NW

            @pl.when(ci < NCHE)
            def _():
                eb = ci * GB
                pltpu.sync_copy(kk_h.at[pl.ds(eb, GB)], kkb)
                pltpu.sync_copy(jj_h.at[pl.ds(eb, GB)], jjb)
                for off, out_h, msk_h in ((0, ixg_h, m1_h), (2, jxg_h, m2_h)):
                    def mkq(gi, _):
                        kv = kkb[pl.ds(gi * 16, 16)]
                        jv = jjb[pl.ds(gi * 16, 16)]
                        qb[pl.ds(gi * 16, 16)] = kv * F + jv + off
                        return 0
                    lax.fori_loop(0, GB // 16, mkq, 0)
                    pltpu.async_copy(tab_h.at[qb], rb, sem).wait()

                    def remap(gi, _):
                        v = rb[pl.ds(gi * 16, 16)]
                        hit = v >= 0
                        mb[pl.ds(gi * 16, 16)] = jnp.where(hit, 1.0, 0.0)
                        rb[pl.ds(gi * 16, 16)] = jnp.where(
                            hit, v, eb + gi * 16 + _iota16())
                        return 0
                    lax.fori_loop(0, GB // 16, remap, 0)
                    pltpu.sync_copy(rb, out_h.at[pl.ds(eb, GB)])
                    pltpu.sync_copy(mb, msk_h.at[pl.ds(eb, GB)])
            return 0
        lax.fori_loop(0, ITER_W, it, 0)

    return k(kk, jj, tab)


def _sc_gather_rows(src, idx):
    @functools.partial(
        pl.kernel, mesh=_mesh(),
        out_type=jax.ShapeDtypeStruct((E, D), jnp.float32),
        scratch_types=[pltpu.VMEM((GB,), jnp.int32),
                       pltpu.VMEM((GB,), jnp.int32),
                       pltpu.VMEM((GB, D), jnp.float32),
                       pltpu.VMEM((GB, D), jnp.float32),
                       pltpu.SemaphoreType.DMA,
                       pltpu.SemaphoreType.DMA],
    )
    def k(src_h, idx_h, out_h, ib0, ib1, r0, r1, s0, s1):
        w = _wid()
        bufs = ((ib0, r0, s0), (ib1, r1, s1))

        for t in range(ITER_W + 1):
            if t < ITER_W:
                ib, rr, ss = bufs[t % 2]
                ci = w + t * NW

                @pl.when(ci < NCHE)
                def _(ib=ib, rr=rr, ss=ss, ci=ci):
                    pltpu.sync_copy(idx_h.at[pl.ds(ci * GB, GB)], ib)
                    pltpu.async_copy(src_h.at[ib], rr, ss)
            if t >= 1:
                ib, rr, ss = bufs[(t - 1) % 2]
                cj = w + (t - 1) * NW

                @pl.when(cj < NCHE)
                def _(ib=ib, rr=rr, ss=ss, cj=cj):
                    pltpu.make_async_copy(src_h.at[ib], rr, ss).wait()
                    pltpu.sync_copy(rr, out_h.at[pl.ds(cj * GB, GB)])

    return k(src, idx)


def _sc_segment_sums(seg, ex, pv, ch_size, nchunks, out_rows):
    H = ch_size // NW
    HP = ((H + 15) // 16) * 16
    TROWS = H + 16
    SW = 2000
    NWIN = E // SW
    CAP = SW + 112
    FB = 64
    padbase = nchunks * ch_size

    @functools.partial(
        pl.kernel, mesh=_mesh(),
        out_type=jax.ShapeDtypeStruct((2 * out_rows, D), jnp.float32),
        scratch_types=[pltpu.VMEM((SW,), jnp.int32),
                       pltpu.VMEM((CAP,), jnp.int32),
                       pltpu.VMEM((CAP,), jnp.int32),
                       pltpu.VMEM((FB,), jnp.int32),
                       pltpu.VMEM((FB + 16,), jnp.int32),
                       pltpu.VMEM((FB, D), jnp.float32),
                       pltpu.VMEM((TROWS, D), jnp.float32),
                       pltpu.VMEM((HP,), jnp.int32),
                       pltpu.SemaphoreType.DMA],
        compiler_params=pltpu.CompilerParams(needs_layout_passes=False),
    )
    def k(seg_h, ex_h, pv_h, out_h, segw, eidx, lrow, gbuf, lbuf, rbuf,
          tab, idxout, sem):
        w = _wid()
        iota = _iota16()

        for a, val_h in ((0, ex_h), (1, pv_h)):
            def per_chunk(ch, _):
                base = ch * ch_size

                def zrow(r, _):
                    for c in range(D // 16):
                        tab[r, pl.ds(c * 16, 16)] = jnp.zeros((16,), jnp.float32)
                    return 0
                lax.fori_loop(0, TROWS, zrow, 0)

                def win(wi, _):
                    pltpu.sync_copy(seg_h.at[pl.ds(wi * SW, SW)], segw)

                    def grp(gi, off):
                        sv = segw[pl.ds(gi * 16, 16)]
                        lo = sv - base
                        mine = (lo >= 0) & (lo < ch_size) & (sv % NW == w)
                        lr = jnp.clip(lo, 0, ch_size - 1) // NW
                        ev = wi * SW + gi * 16 + iota
                        plsc.store_compressed(eidx.at[pl.ds(off, 16)], ev, mask=mine)
                        plsc.store_compressed(lrow.at[pl.ds(off, 16)], lr, mask=mine)
                        return off + jnp.sum(mine.astype(jnp.int32))
                    off = lax.fori_loop(0, SW // 16, grp, 0)

                    ones = jnp.ones((16,), jnp.bool_)
                    for pg in range(4):
                        plsc.store_compressed(eidx.at[pl.ds(off + pg * 16, 16)],
                                              iota, mask=ones)
                        plsc.store_compressed(lrow.at[pl.ds(off + pg * 16, 16)],
                                              H + iota, mask=ones)

                    def flush(b, _):
                        @pl.when(b * FB < off)
                        def _():
                            def cp(i, _):
                                gbuf[pl.ds(i * 16, 16)] = eidx[
                                    pl.ds(b * FB + i * 16, 16)]
                                lbuf[pl.ds(i * 16, 16)] = lrow[
                                    pl.ds(b * FB + i * 16, 16)]
                                return 0
                            lax.fori_loop(0, FB // 16, cp, 0)
                            pltpu.async_copy(val_h.at[gbuf], rbuf, sem).wait()

                            def acc(r, _):
                                s = lbuf[pl.ds(r, 16)][0]
                                for dpart in range(D // 16):
                                    plsc.addupdate(
                                        tab.at[s, pl.ds(dpart * 16, 16)],
                                        rbuf[r, pl.ds(dpart * 16, 16)])
                                return 0
                            lax.fori_loop(0, FB, acc, 0)
                        return 0
                    lax.fori_loop(0, SW // FB, flush, 0)
                    return 0
                lax.fori_loop(0, NWIN, win, 0)

                def mkout(gi, _):
                    j = gi * 16 + iota
                    inr = j < H
                    oidx = jnp.where(
                        inr, a * out_rows + base + j * NW + w,
                        a * out_rows + padbase + (w % 6) * 16
                        + jnp.clip(j - H, 0, 15))
                    idxout[pl.ds(gi * 16, 16)] = oidx
                    return 0
                lax.fori_loop(0, HP // 16, mkout, 0)
                pltpu.sync_copy(tab.at[pl.ds(0, HP)], out_h.at[idxout])
                return 0
            lax.fori_loop(0, nchunks, per_chunk, 0)

    out = k(seg, ex, pv)
    return out.reshape(2, out_rows, D)


def kernel(net, inp, corr, flow, ii, jj, kk, params):
    del flow
    net2, inp2, corr2 = net[0], inp[0], corr[0]

    x = _tc_embed(net2, inp2, corr2, params)

    tab = _sc_neigh_table(kk, jj)
    ixg, jxg, m1, m2 = _sc_neigh_query(kk, jj, tab)
    gx1 = _sc_gather_rows(x, ixg)
    gx2 = _sc_gather_rows(x, jxg)
    x2 = _tc_resid(x, gx1, gx2, m1, m2, params)

    ex, p = _tc_softprep(x2, None, params["agg_kk"])
    parts = _sc_segment_sums(kk, ex, p, 4000, 1, 4096)
    h_kk = _tc_seg_h(parts, params["agg_kk"])
    hg1 = _sc_gather_rows(h_kk, kk)

    seg_ij = ii * NF + jj
    x3, ex2, p2 = _tc_softprep(x2, hg1, params["agg_ij"])
    parts2 = _sc_segment_sums(seg_ij, ex2, p2, 7200, 2, 14848)
    h_ij = _tc_seg_h(parts2, params["agg_ij"])
    hg2 = _sc_gather_rows(h_ij, seg_ij)

    net_o, d_o, w_o = _tc_final(x3, hg2, params)
    return net_o[None], d_o[None], w_o[None]


# speedup vs baseline: 2.4909x; 1.0234x over previous
import functools

import jax
import jax.numpy as jnp
from jax import lax
from jax.experimental import pallas as pl
from jax.experimental.pallas import tpu as pltpu
from jax.experimental.pallas import tpu_sc as plsc

E = 40000
D = 384
CD = 882
NK = 4000
NF = 120
F = NF + 2
NW = 32
SLOTS_W = 15256
TPAD = NW * SLOTS_W
LOCPAD = 15264

BE = 1000
GB = 160
NCHE = E // GB
ITER_W = (NCHE + NW - 1) // NW

SEGC = 2048
PADR = 48
ZR = SEGC // 16
GBS = 64
NCHS = E // GBS
ITER_S = (NCHS + NW - 1) // NW


def _f32(x):
    return x.astype(jnp.float32)


def _ln(x, g, b, eps=1e-3):
    mu = jnp.mean(x, axis=-1, keepdims=True)
    var = jnp.mean((x - mu) ** 2, axis=-1, keepdims=True)
    return (x - mu) / jnp.sqrt(var + eps) * g + b


def _mm(x, w):
    return jnp.dot(x, w, preferred_element_type=jnp.float32)


def _relu(x):
    return jnp.maximum(x, 0.0)


def _full(shape):
    return pl.BlockSpec(shape, lambda i: tuple(0 for _ in shape))


def _eblk(d):
    return pl.BlockSpec((BE, d), lambda i: (i, 0))


def _tc_embed(net, inp, corr, p):
    c = p["corr"]

    def body(net_r, inp_r, corr_r, w1, b1, w2, b2, lg, lb, w3, b3, ng, nb, o_r):
        h = _relu(_mm(corr_r[...], w1[...]) + b1[...])
        h = _mm(h, w2[...]) + b2[...]
        h = _relu(_ln(h, lg[...], lb[...]))
        ce = _mm(h, w3[...]) + b3[...]
        o_r[...] = _ln(net_r[...] + inp_r[...] + ce, ng[...], nb[...])

    return pl.pallas_call(
        body,
        grid=(E // BE,),
        in_specs=[_eblk(D), _eblk(D), _eblk(CD),
                  _full((CD, D)), _full((1, D)), _full((D, D)), _full((1, D)),
                  _full((1, D)), _full((1, D)), _full((D, D)), _full((1, D)),
                  _full((1, D)), _full((1, D))],
        out_specs=_eblk(D),
        out_shape=jax.ShapeDtypeStruct((E, D), jnp.float32),
    )(net, inp, corr,
      c["l1"]["w"].T, c["l1"]["b"][None], c["l2"]["w"].T, c["l2"]["b"][None],
      c["ln"]["g"][None], c["ln"]["b"][None], c["l3"]["w"].T, c["l3"]["b"][None],
      p["norm"]["g"][None], p["norm"]["b"][None])


def _tc_resid(x, g1, g2, m1, m2, p):
    c1a, c1b = p["c1"]
    c2a, c2b = p["c2"]

    def body(x_r, g1_r, g2_r, m1_r, m2_r,
             aw1, ab1, aw2, ab2, bw1, bb1, bw2, bb2, o_r):
        a1 = g1_r[...] * m1_r[...]
        a2 = g2_r[...] * m2_r[...]
        r1 = _mm(_relu(_mm(a1, aw1[...]) + ab1[...]), aw2[...]) + ab2[...]
        r2 = _mm(_relu(_mm(a2, bw1[...]) + bb1[...]), bw2[...]) + bb2[...]
        o_r[...] = x_r[...] + r1 + r2

    return pl.pallas_call(
        body,
        grid=(E // BE,),
        in_specs=[_eblk(D), _eblk(D), _eblk(D), _eblk(1), _eblk(1)]
        + [_full((D, D)), _full((1, D))] * 4,
        out_specs=_eblk(D),
        out_shape=jax.ShapeDtypeStruct((E, D), jnp.float32),
    )(x, g1, g2, m1[:, None], m2[:, None],
      c1a["w"].T, c1a["b"][None], c1b["w"].T, c1b["b"][None],
      c2a["w"].T, c2a["b"][None], c2b["w"].T, c2b["b"][None])


def _tc_softprep(x, hg, agg):
    add_h = hg is not None

    def body(*refs):
        if add_h:
            x_r, hg_r, fw, fb, gw, gb, x3_r, ex_r, p_r = refs
            x3 = x_r[...] + hg_r[...]
            x3_r[...] = x3
        else:
            x_r, fw, fb, gw, gb, ex_r, p_r = refs
            x3 = x_r[...]
        ex = jnp.exp(_mm(x3, gw[...]) + gb[...])
        ex_r[...] = ex
        p_r[...] = (_mm(x3, fw[...]) + fb[...]) * ex

    n_out = 3 if add_h else 2
    ins = [x, hg] if add_h else [x]
    return pl.pallas_call(
        body,
        grid=(E // BE,),
        in_specs=[_eblk(D)] * len(ins)
        + [_full((D, D)), _full((1, D)), _full((D, D)), _full((1, D))],
        out_specs=[_eblk(D)] * n_out,
        out_shape=[jax.ShapeDtypeStruct((E, D), jnp.float32)] * n_out,
    )(*ins, agg["f"]["w"].T, agg["f"]["b"][None],
      agg["g"]["w"].T, agg["g"]["b"][None])


def _tc_seg_h(parts, agg):
    R = parts.shape[1]
    BD = 512

    def body(p_r, hw, hb, o_r):
        blk = p_r[...]
        es = blk[0]
        ps = blk[1]
        y = ps / jnp.where(es == 0.0, 1.0, es)
        o_r[...] = _mm(y, hw[...]) + hb[...]

    return pl.pallas_call(
        body,
        grid=(R // BD,),
        in_specs=[pl.BlockSpec((2, BD, D), lambda i: (0, i, 0)),
                  _full((D, D)), _full((1, D))],
        out_specs=pl.BlockSpec((BD, D), lambda i: (i, 0)),
        out_shape=jax.ShapeDtypeStruct((R, D), jnp.float32),
    )(parts, agg["h"]["w"].T, agg["h"]["b"][None])


def _tc_final(x, hg, p):
    g = p["gru"]
    g1, g2 = g["gr1"], g["gr2"]

    def body(x_r, hg_r, l1g, l1b, g1g, g1gb, g1r1, g1r1b, g1r2, g1r2b,
             l2g, l2b, g2g, g2gb, g2r1, g2r1b, g2r2, g2r2b,
             dw, db, ww, wb, net_r, d_r, w_r):
        n = _ln(x_r[...] + hg_r[...], l1g[...], l1b[...])
        gate = jax.nn.sigmoid(_mm(n, g1g[...]) + g1gb[...])
        res = _mm(_relu(_mm(n, g1r1[...]) + g1r1b[...]), g1r2[...]) + g1r2b[...]
        n = n * gate + res
        n = _ln(n, l2g[...], l2b[...])
        gate = jax.nn.sigmoid(_mm(n, g2g[...]) + g2gb[...])
        res = _mm(_relu(_mm(n, g2r1[...]) + g2r1b[...]), g2r2[...]) + g2r2b[...]
        n = n * gate + res
        net_r[...] = n
        r = _relu(n)
        d_r[...] = _mm(r, dw[...]) + db[...]
        w_r[...] = jax.nn.sigmoid(_mm(r, ww[...]) + wb[...])

    return pl.pallas_call(
        body,
        grid=(E // BE,),
        in_specs=[_eblk(D), _eblk(D),
                  _full((1, D)), _full((1, D)),
                  _full((D, D)), _full((1, D)), _full((D, D)), _full((1, D)),
                  _full((D, D)), _full((1, D)),
                  _full((1, D)), _full((1, D)),
                  _full((D, D)), _full((1, D)), _full((D, D)), _full((1, D)),
                  _full((D, D)), _full((1, D)),
                  _full((D, 2)), _full((1, 2)), _full((D, 2)), _full((1, 2))],
        out_specs=[_eblk(D), pl.BlockSpec((BE, 2), lambda i: (i, 0)),
                   pl.BlockSpec((BE, 2), lambda i: (i, 0))],
        out_shape=[jax.ShapeDtypeStruct((E, D), jnp.float32),
                   jax.ShapeDtypeStruct((E, 2), jnp.float32),
                   jax.ShapeDtypeStruct((E, 2), jnp.float32)],
    )(x, hg,
      g["ln1"]["g"][None], g["ln1"]["b"][None],
      g1["gate"]["w"].T, g1["gate"]["b"][None],
      g1["res1"]["w"].T, g1["res1"]["b"][None],
      g1["res2"]["w"].T, g1["res2"]["b"][None],
      g["ln2"]["g"][None], g["ln2"]["b"][None],
      g2["gate"]["w"].T, g2["gate"]["b"][None],
      g2["res1"]["w"].T, g2["res1"]["b"][None],
      g2["res2"]["w"].T, g2["res2"]["b"][None],
      p["d"]["w"].T, p["d"]["b"][None],
      p["w"]["w"].T, p["w"]["b"][None])


def _mesh():
    return plsc.VectorSubcoreMesh(core_axis_name="c", subcore_axis_name="s")


def _wid():
    return lax.axis_index("s") * 2 + lax.axis_index("c")


def _iota16():
    return lax.broadcasted_iota(jnp.int32, (16,), 0)


def _sc_neigh_table(kk, jj):
    CH = 2000

    @functools.partial(
        pl.kernel, mesh=_mesh(),
        out_type=jax.ShapeDtypeStruct((TPAD,), jnp.int32),
        scratch_types=[pltpu.VMEM((CH,), jnp.int32),
                       pltpu.VMEM((CH,), jnp.int32),
                       pltpu.VMEM((LOCPAD,), jnp.int32)],
        compiler_params=pltpu.CompilerParams(needs_layout_passes=False),
    )
    def k(kk_h, jj_h, tab_h, kkb, jjb, loc):
        base = _wid() * SLOTS_W

        def init(i, _):
            loc[pl.ds(i * 16, 16)] = jnp.full((16,), -1, jnp.int32)
            return 0
        lax.fori_loop(0, LOCPAD // 16, init, 0)

        def chunk(ci, _):
            pltpu.sync_copy(kk_h.at[pl.ds(ci * CH, CH)], kkb)
            pltpu.sync_copy(jj_h.at[pl.ds(ci * CH, CH)], jjb)

            def grp(gi, _):
                kv = kkb[pl.ds(gi * 16, 16)]
                jv = jjb[pl.ds(gi * 16, 16)]
                lo = kv * F + jv + 1 - base
                valid = (lo >= 0) & (lo < SLOTS_W)
                loc_c = jnp.clip(lo, 0, SLOTS_W - 1)
                ev = ci * CH + gi * 16 + _iota16()
                plsc.store_scatter(loc, [loc_c], ev, mask=valid)
                return 0
            lax.fori_loop(0, CH // 16, grp, 0)
            return 0
        lax.fori_loop(0, E // CH, chunk, 0)
        pltpu.sync_copy(loc.at[pl.ds(0, SLOTS_W)], tab_h.at[pl.ds(base, SLOTS_W)])

    return k(kk, jj)


def _sc_neigh_query(kk, jj, tab):
    @functools.partial(
        pl.kernel, mesh=_mesh(),
        out_type=[jax.ShapeDtypeStruct((E,), jnp.int32),
                  jax.ShapeDtypeStruct((E,), jnp.int32),
                  jax.ShapeDtypeStruct((E,), jnp.float32),
                  jax.ShapeDtypeStruct((E,), jnp.float32)],
        scratch_types=[pltpu.VMEM((GB,), jnp.int32),
                       pltpu.VMEM((GB,), jnp.int32),
                       pltpu.VMEM((GB,), jnp.int32),
                       pltpu.VMEM((GB,), jnp.int32),
                       pltpu.VMEM((GB,), jnp.float32),
                       pltpu.SemaphoreType.DMA],
    )
    def k(kk_h, jj_h, tab_h, ixg_h, jxg_h, m1_h, m2_h, kkb, jjb, qb, rb,
          mb, sem):
        w = _wid()

        def it(t, _):
            ci = w + t * NW

            @pl.when(ci < NCHE)
            def _():
                eb = ci * GB
                pltpu.sync_copy(kk_h.at[pl.ds(eb, GB)], kkb)
                pltpu.sync_copy(jj_h.at[pl.ds(eb, GB)], jjb)
                for off, out_h, msk_h in ((0, ixg_h, m1_h), (2, jxg_h, m2_h)):
                    def mkq(gi, _):
                        kv = kkb[pl.ds(gi * 16, 16)]
                        jv = jjb[pl.ds(gi * 16, 16)]
                        qb[pl.ds(gi * 16, 16)] = kv * F + jv + off
                        return 0
                    lax.fori_loop(0, GB // 16, mkq, 0)
                    pltpu.async_copy(tab_h.at[qb], rb, sem).wait()

                    def remap(gi, _):
                        v = rb[pl.ds(gi * 16, 16)]
                        hit = v >= 0
                        mb[pl.ds(gi * 16, 16)] = jnp.where(hit, 1.0, 0.0)
                        rb[pl.ds(gi * 16, 16)] = jnp.where(
                            hit, v, eb + gi * 16 + _iota16())
                        return 0
                    lax.fori_loop(0, GB // 16, remap, 0)
                    pltpu.sync_copy(rb, out_h.at[pl.ds(eb, GB)])
                    pltpu.sync_copy(mb, msk_h.at[pl.ds(eb, GB)])
            return 0
        lax.fori_loop(0, ITER_W, it, 0)

    return k(kk, jj, tab)


def _sc_gather_rows(src, idx):
    @functools.partial(
        pl.kernel, mesh=_mesh(),
        out_type=jax.ShapeDtypeStruct((E, D), jnp.float32),
        scratch_types=[pltpu.VMEM((GB,), jnp.int32),
                       pltpu.VMEM((GB,), jnp.int32),
                       pltpu.VMEM((GB, D), jnp.float32),
                       pltpu.VMEM((GB, D), jnp.float32),
                       pltpu.SemaphoreType.DMA,
                       pltpu.SemaphoreType.DMA],
    )
    def k(src_h, idx_h, out_h, ib0, ib1, r0, r1, s0, s1):
        w = _wid()
        bufs = ((ib0, r0, s0), (ib1, r1, s1))

        for t in range(ITER_W + 1):
            if t < ITER_W:
                ib, rr, ss = bufs[t % 2]
                ci = w + t * NW

                @pl.when(ci < NCHE)
                def _(ib=ib, rr=rr, ss=ss, ci=ci):
                    pltpu.sync_copy(idx_h.at[pl.ds(ci * GB, GB)], ib)
                    pltpu.async_copy(src_h.at[ib], rr, ss)
            if t >= 1:
                ib, rr, ss = bufs[(t - 1) % 2]
                cj = w + (t - 1) * NW

                @pl.when(cj < NCHE)
                def _(ib=ib, rr=rr, ss=ss, cj=cj):
                    pltpu.make_async_copy(src_h.at[ib], rr, ss).wait()
                    pltpu.sync_copy(rr, out_h.at[pl.ds(cj * GB, GB)])

    return k(src, idx)


def _sc_segment_sums(seg, ex, pv, ch_size, nchunks, out_rows, fb):
    H = ch_size // NW
    HP = ((H + 15) // 16) * 16
    TROWS = H + 16
    SW = 4000
    NWIN = E // SW
    FB = fb
    CAP = SW + FB
    padbase = nchunks * ch_size

    @functools.partial(
        pl.kernel, mesh=_mesh(),
        out_type=jax.ShapeDtypeStruct((2 * out_rows, D), jnp.float32),
        scratch_types=[pltpu.VMEM((SW,), jnp.int32),
                       pltpu.VMEM((CAP,), jnp.int32),
                       pltpu.VMEM((CAP,), jnp.int32),
                       pltpu.VMEM((FB,), jnp.int32),
                       pltpu.VMEM((FB + 16,), jnp.int32),
                       pltpu.VMEM((FB, D), jnp.float32),
                       pltpu.VMEM((TROWS, D), jnp.float32),
                       pltpu.VMEM((HP,), jnp.int32),
                       pltpu.SemaphoreType.DMA],
        compiler_params=pltpu.CompilerParams(needs_layout_passes=False),
    )
    def k(seg_h, ex_h, pv_h, out_h, segw, eidx, lrow, gbuf, lbuf, rbuf,
          tab, idxout, sem):
        w = _wid()
        iota = _iota16()

        for a, val_h in ((0, ex_h), (1, pv_h)):
            def per_chunk(ch, _):
                base = ch * ch_size

                def zrow(r, _):
                    for c in range(D // 16):
                        tab[r, pl.ds(c * 16, 16)] = jnp.zeros((16,), jnp.float32)
                    return 0
                lax.fori_loop(0, TROWS, zrow, 0)

                def win(wi, _):
                    pltpu.sync_copy(seg_h.at[pl.ds(wi * SW, SW)], segw)

                    def grp(gi, off):
                        sv = segw[pl.ds(gi * 16, 16)]
                        lo = sv - base
                        mine = (lo >= 0) & (lo < ch_size) & (sv % NW == w)
                        lr = jnp.clip(lo, 0, ch_size - 1) // NW
                        ev = wi * SW + gi * 16 + iota
                        plsc.store_compressed(eidx.at[pl.ds(off, 16)], ev, mask=mine)
                        plsc.store_compressed(lrow.at[pl.ds(off, 16)], lr, mask=mine)
                        return off + jnp.sum(mine.astype(jnp.int32))
                    off = lax.fori_loop(0, SW // 16, grp, 0)

                    ones = jnp.ones((16,), jnp.bool_)
                    for pg in range(FB // 16):
                        plsc.store_compressed(eidx.at[pl.ds(off + pg * 16, 16)],
                                              iota, mask=ones)
                        plsc.store_compressed(lrow.at[pl.ds(off + pg * 16, 16)],
                                              H + iota, mask=ones)

                    def flush(b, _):
                        @pl.when(b * FB < off)
                        def _():
                            def cp(i, _):
                                gbuf[pl.ds(i * 16, 16)] = eidx[
                                    pl.ds(b * FB + i * 16, 16)]
                                lbuf[pl.ds(i * 16, 16)] = lrow[
                                    pl.ds(b * FB + i * 16, 16)]
                                return 0
                            lax.fori_loop(0, FB // 16, cp, 0)
                            pltpu.async_copy(val_h.at[gbuf], rbuf, sem).wait()

                            def acc(r, _):
                                s = lbuf[pl.ds(r, 16)][0]
                                for dpart in range(D // 16):
                                    plsc.addupdate(
                                        tab.at[s, pl.ds(dpart * 16, 16)],
                                        rbuf[r, pl.ds(dpart * 16, 16)])
                                return 0
                            lax.fori_loop(0, FB, acc, 0)
                        return 0
                    lax.fori_loop(0, SW // FB, flush, 0)
                    return 0
                lax.fori_loop(0, NWIN, win, 0)

                def mkout(gi, _):
                    j = gi * 16 + iota
                    inr = j < H
                    oidx = jnp.where(
                        inr, a * out_rows + base + j * NW + w,
                        a * out_rows + padbase + (w % 6) * 16
                        + jnp.clip(j - H, 0, 15))
                    idxout[pl.ds(gi * 16, 16)] = oidx
                    return 0
                lax.fori_loop(0, HP // 16, mkout, 0)
                pltpu.sync_copy(tab.at[pl.ds(0, HP)], out_h.at[idxout])
                return 0
            lax.fori_loop(0, nchunks, per_chunk, 0)

    out = k(seg, ex, pv)
    return out.reshape(2, out_rows, D)


def kernel(net, inp, corr, flow, ii, jj, kk, params):
    del flow
    net2, inp2, corr2 = net[0], inp[0], corr[0]

    x = _tc_embed(net2, inp2, corr2, params)

    tab = _sc_neigh_table(kk, jj)
    ixg, jxg, m1, m2 = _sc_neigh_query(kk, jj, tab)
    gx1 = _sc_gather_rows(x, ixg)
    gx2 = _sc_gather_rows(x, jxg)
    x2 = _tc_resid(x, gx1, gx2, m1, m2, params)

    ex, p = _tc_softprep(x2, None, params["agg_kk"])
    parts = _sc_segment_sums(kk, ex, p, 4000, 1, 4096, 128)
    h_kk = _tc_seg_h(parts, params["agg_kk"])
    hg1 = _sc_gather_rows(h_kk, kk)

    seg_ij = ii * NF + jj
    x3, ex2, p2 = _tc_softprep(x2, hg1, params["agg_ij"])
    parts2 = _sc_segment_sums(seg_ij, ex2, p2, 4800, 3, 14848, 64)
    h_ij = _tc_seg_h(parts2, params["agg_ij"])
    hg2 = _sc_gather_rows(h_ij, seg_ij)

    net_o, d_o, w_o = _tc_final(x3, hg2, params)
    return net_o[None], d_o[None], w_o[None]
